# initial kernel scaffold (unmeasured)
import jax
import jax.numpy as jnp
from jax import lax
from jax.experimental import pallas as pl
from jax.experimental.pallas import tpu as pltpu

N_DEV = 8
B, SQ, SKV, DH = 2, 128, 128, 64
H_LOC = 4
CTX_LOC = H_LOC * DH
ROWS = B * SQ
D_MODEL = 512
BLK = 64


def kernel(x, Wq, K_ext, V_ext, Wo):
    my = lax.axis_index("i")

    x2 = x.reshape(ROWS, D_MODEL)
    wq_loc = lax.dynamic_slice(Wq, (0, my * CTX_LOC), (D_MODEL, CTX_LOC))
    k_t = K_ext.transpose(0, 2, 1, 3)
    v_t = V_ext.transpose(0, 2, 1, 3)

    def body(x_ref, wq_ref, k_ref, v_ref, wo_ref, out_ref,
             comm_ref, acc_ref, send_sems, recv_sems):
        left = lax.rem(my + N_DEV - 1, N_DEV)
        right = lax.rem(my + 1, N_DEV)

        barrier = pltpu.get_barrier_semaphore()
        for nbr in (left, right):
            pl.semaphore_signal(
                barrier, inc=1,
                device_id=(nbr,), device_id_type=pl.DeviceIdType.MESH,
            )
        pl.semaphore_wait(barrier, 2)

        xb = x_ref[...].astype(jnp.bfloat16)
        wqb = wq_ref[...].astype(jnp.bfloat16)
        q = jnp.dot(xb, wqb, preferred_element_type=jnp.float32)

        qb = lax.broadcasted_iota(jnp.int32, (SQ, SKV), 0) // BLK
        kb = lax.broadcasted_iota(jnp.int32, (SQ, SKV), 1) // BLK
        mask = (qb == kb) | (kb == 0) | (lax.rem(qb + kb, 3) == 0)

        for b in range(B):
            for h in range(H_LOC):
                qbh = q[b * SQ:(b + 1) * SQ, h * DH:(h + 1) * DH]
                kbh = k_ref[b, h, :, :].astype(jnp.bfloat16)
                s = lax.dot_general(
                    qbh.astype(jnp.bfloat16), kbh,
                    (((1,), (1,)), ((), ())),
                    preferred_element_type=jnp.float32,
                ) * 0.125
                s = jnp.where(mask, s, -1e9)
                s_max = jnp.max(s, axis=1, keepdims=True)
                w = jnp.exp(s - s_max)
                w = w / jnp.sum(w, axis=1, keepdims=True)
                vbh = v_ref[b, h, :, :].astype(jnp.bfloat16)
                ctx = jnp.dot(w.astype(jnp.bfloat16), vbh,
                              preferred_element_type=jnp.float32)
                comm_ref[0, b * SQ:(b + 1) * SQ, h * DH:(h + 1) * DH] = (
                    ctx.astype(jnp.bfloat16))

        wo_loc = wo_ref[pl.ds(my * CTX_LOC, CTX_LOC), :].astype(jnp.bfloat16)
        acc_ref[...] = jnp.dot(comm_ref[0], wo_loc,
                               preferred_element_type=jnp.float32)

        for h in range(N_DEV - 1):
            rdma = pltpu.make_async_remote_copy(
                src_ref=comm_ref.at[h],
                dst_ref=comm_ref.at[h + 1],
                send_sem=send_sems.at[h],
                recv_sem=recv_sems.at[h],
                device_id=(right,),
                device_id_type=pl.DeviceIdType.MESH,
            )
            rdma.start()
            rdma.wait()
            origin = lax.rem(my - (h + 1) + 2 * N_DEV, N_DEV)
            wo_o = wo_ref[pl.ds(origin * CTX_LOC, CTX_LOC), :].astype(
                jnp.bfloat16)
            acc_ref[...] += jnp.dot(comm_ref[h + 1], wo_o,
                                    preferred_element_type=jnp.float32)

        out_ref[...] = acc_ref[...]

    out = pl.pallas_call(
        body,
        out_shape=jax.ShapeDtypeStruct((ROWS, D_MODEL), jnp.float32),
        in_specs=[pl.BlockSpec(memory_space=pltpu.VMEM)] * 5,
        out_specs=pl.BlockSpec(memory_space=pltpu.VMEM),
        scratch_shapes=[
            pltpu.VMEM((N_DEV, ROWS, CTX_LOC), jnp.bfloat16),
            pltpu.VMEM((ROWS, D_MODEL), jnp.float32),
            pltpu.SemaphoreType.DMA((N_DEV - 1,)),
            pltpu.SemaphoreType.DMA((N_DEV - 1,)),
        ],
        compiler_params=pltpu.CompilerParams(collective_id=0),
    )(x2, wq_loc, k_t, v_t, Wo)
    return out.reshape(B, SQ, D_MODEL)


# baseline (device time: 35548 ns/iter reference)
import jax
import jax.numpy as jnp
from jax import lax
from jax.experimental import pallas as pl
from jax.experimental.pallas import tpu as pltpu

N_DEV = 8
B, SQ, SKV, DH = 2, 128, 128, 64
H_LOC = 4
CTX_LOC = H_LOC * DH
ROWS = B * SQ
D_MODEL = 512
BLK = 64


def kernel(x, Wq, K_ext, V_ext, Wo):
    my = lax.axis_index("i")

    x2 = x.reshape(ROWS, D_MODEL)
    wq_loc = lax.dynamic_slice(Wq, (0, my * CTX_LOC), (D_MODEL, CTX_LOC))
    k_t = K_ext.transpose(0, 2, 1, 3)
    v_t = V_ext.transpose(0, 2, 1, 3)

    def body(x_ref, wq_ref, k_ref, v_ref, wo_ref, out_ref,
             comm_ref, acc_ref, send_sems, recv_sems):
        my = lax.axis_index("i")
        left = lax.rem(my + N_DEV - 1, N_DEV)
        right = lax.rem(my + 1, N_DEV)

        barrier = pltpu.get_barrier_semaphore()
        for nbr in (left, right):
            pl.semaphore_signal(
                barrier, inc=1,
                device_id=(nbr,), device_id_type=pl.DeviceIdType.MESH,
            )
        pl.semaphore_wait(barrier, 2)

        xb = x_ref[...].astype(jnp.bfloat16)
        wqb = wq_ref[...].astype(jnp.bfloat16)
        q = jnp.dot(xb, wqb, preferred_element_type=jnp.float32)

        qb = lax.broadcasted_iota(jnp.int32, (SQ, SKV), 0) // BLK
        kb = lax.broadcasted_iota(jnp.int32, (SQ, SKV), 1) // BLK
        mask = (qb == kb) | (kb == 0) | (lax.rem(qb + kb, 3) == 0)

        for b in range(B):
            for h in range(H_LOC):
                qbh = q[b * SQ:(b + 1) * SQ, h * DH:(h + 1) * DH]
                kbh = k_ref[b, h, :, :].astype(jnp.bfloat16)
                s = lax.dot_general(
                    qbh.astype(jnp.bfloat16), kbh,
                    (((1,), (1,)), ((), ())),
                    preferred_element_type=jnp.float32,
                ) * 0.125
                s = jnp.where(mask, s, -1e9)
                s_max = jnp.max(s, axis=1, keepdims=True)
                w = jnp.exp(s - s_max)
                w = w / jnp.sum(w, axis=1, keepdims=True)
                vbh = v_ref[b, h, :, :].astype(jnp.bfloat16)
                ctx = jnp.dot(w.astype(jnp.bfloat16), vbh,
                              preferred_element_type=jnp.float32)
                comm_ref[0, b * SQ:(b + 1) * SQ, h * DH:(h + 1) * DH] = (
                    ctx.astype(jnp.bfloat16))

        wo_loc = wo_ref[pl.ds(my * CTX_LOC, CTX_LOC), :].astype(jnp.bfloat16)
        acc_ref[...] = jnp.dot(comm_ref[0], wo_loc,
                               preferred_element_type=jnp.float32)

        for h in range(N_DEV - 1):
            rdma = pltpu.make_async_remote_copy(
                src_ref=comm_ref.at[h],
                dst_ref=comm_ref.at[h + 1],
                send_sem=send_sems.at[h],
                recv_sem=recv_sems.at[h],
                device_id=(right,),
                device_id_type=pl.DeviceIdType.MESH,
            )
            rdma.start()
            rdma.wait()
            origin = lax.rem(my - (h + 1) + 2 * N_DEV, N_DEV)
            wo_o = wo_ref[pl.ds(origin * CTX_LOC, CTX_LOC), :].astype(
                jnp.bfloat16)
            acc_ref[...] += jnp.dot(comm_ref[h + 1], wo_o,
                                    preferred_element_type=jnp.float32)

        out_ref[...] = acc_ref[...]

    out = pl.pallas_call(
        body,
        out_shape=jax.ShapeDtypeStruct((ROWS, D_MODEL), jnp.float32),
        in_specs=[pl.BlockSpec(memory_space=pltpu.VMEM)] * 5,
        out_specs=pl.BlockSpec(memory_space=pltpu.VMEM),
        scratch_shapes=[
            pltpu.VMEM((N_DEV, ROWS, CTX_LOC), jnp.bfloat16),
            pltpu.VMEM((ROWS, D_MODEL), jnp.float32),
            pltpu.SemaphoreType.DMA((N_DEV - 1,)),
            pltpu.SemaphoreType.DMA((N_DEV - 1,)),
        ],
        compiler_params=pltpu.CompilerParams(collective_id=0),
    )(x2, wq_loc, k_t, v_t, Wo)
    return out.reshape(B, SQ, D_MODEL)


# device time: 24469 ns/iter; 1.4528x vs baseline; 1.4528x over previous
import functools

import jax
import jax.numpy as jnp
from jax import lax
from jax.experimental import pallas as pl
from jax.experimental.pallas import tpu as pltpu

N_DEV = 8
B, SQ, SKV, DH = 2, 128, 128, 64
H_LOC = 4
CTX_LOC = H_LOC * DH
ROWS = B * SQ
D_MODEL = 512
BLK = 64


def kernel(x, Wq, K_ext, V_ext, Wo):
    my_out = lax.axis_index("i")

    x2 = x.reshape(ROWS, D_MODEL)
    wq_loc = lax.dynamic_slice(Wq, (0, my_out * CTX_LOC), (D_MODEL, CTX_LOC))
    k_t = K_ext.transpose(0, 2, 1, 3)
    v_t = V_ext.transpose(0, 2, 1, 3)

    def body(x_ref, wq_ref, k_ref, v_ref, wo_ref, out_ref,
             own_ref, comm_ref, acc_ref, send_sems, recv_sems):
        my = lax.axis_index("i")
        peers = [lax.rem(my + d, N_DEV) for d in range(1, N_DEV)]

        barrier = pltpu.get_barrier_semaphore()
        for p in peers:
            pl.semaphore_signal(
                barrier, inc=1,
                device_id=(p,), device_id_type=pl.DeviceIdType.MESH,
            )
        pl.semaphore_wait(barrier, N_DEV - 1)

        xb = x_ref[...].astype(jnp.bfloat16)
        wqb = wq_ref[...].astype(jnp.bfloat16)
        q = jnp.dot(xb, wqb, preferred_element_type=jnp.float32)

        qb = lax.broadcasted_iota(jnp.int32, (SQ, SKV), 0) // BLK
        kb = lax.broadcasted_iota(jnp.int32, (SQ, SKV), 1) // BLK
        mask = (qb == kb) | (kb == 0) | (lax.rem(qb + kb, 3) == 0)

        for b in range(B):
            for h in range(H_LOC):
                qbh = q[b * SQ:(b + 1) * SQ, h * DH:(h + 1) * DH]
                kbh = k_ref[b, h, :, :].astype(jnp.bfloat16)
                s = lax.dot_general(
                    qbh.astype(jnp.bfloat16), kbh,
                    (((1,), (1,)), ((), ())),
                    preferred_element_type=jnp.float32,
                ) * 0.125
                s = jnp.where(mask, s, -1e9)
                s_max = jnp.max(s, axis=1, keepdims=True)
                w = jnp.exp(s - s_max)
                w = w / jnp.sum(w, axis=1, keepdims=True)
                vbh = v_ref[b, h, :, :].astype(jnp.bfloat16)
                ctx = jnp.dot(w.astype(jnp.bfloat16), vbh,
                              preferred_element_type=jnp.float32)
                own_ref[b * SQ:(b + 1) * SQ, h * DH:(h + 1) * DH] = (
                    ctx.astype(jnp.bfloat16))

        rdmas = []
        for d in range(1, N_DEV):
            rdma = pltpu.make_async_remote_copy(
                src_ref=own_ref,
                dst_ref=comm_ref.at[my],
                send_sem=send_sems.at[d - 1],
                recv_sem=recv_sems.at[d - 1],
                device_id=(lax.rem(my + d, N_DEV),),
                device_id_type=pl.DeviceIdType.MESH,
            )
            rdma.start()
            rdmas.append(rdma)

        wo_loc = wo_ref[pl.ds(my * CTX_LOC, CTX_LOC), :].astype(jnp.bfloat16)
        acc_ref[...] = jnp.dot(own_ref[...], wo_loc,
                               preferred_element_type=jnp.float32)

        for d in range(1, N_DEV):
            rdmas[d - 1].wait_recv()
            o = lax.rem(my - d + N_DEV, N_DEV)
            wo_o = wo_ref[pl.ds(o * CTX_LOC, CTX_LOC), :].astype(jnp.bfloat16)
            acc_ref[...] += jnp.dot(comm_ref[o], wo_o,
                                    preferred_element_type=jnp.float32)

        out_ref[...] = acc_ref[...]

        for d in range(1, N_DEV):
            rdmas[d - 1].wait_send()

        @functools.partial(
            pl.run_scoped, exit_barrier=pltpu.SemaphoreType.REGULAR)
        def _(exit_barrier):
            for p in peers:
                pl.semaphore_signal(
                    exit_barrier, inc=1,
                    device_id=(p,), device_id_type=pl.DeviceIdType.MESH,
                )
            pl.semaphore_wait(exit_barrier, N_DEV - 1)

    out = pl.pallas_call(
        body,
        out_shape=jax.ShapeDtypeStruct((ROWS, D_MODEL), jnp.float32),
        in_specs=[pl.BlockSpec(memory_space=pltpu.VMEM)] * 5,
        out_specs=pl.BlockSpec(memory_space=pltpu.VMEM),
        scratch_shapes=[
            pltpu.VMEM((ROWS, CTX_LOC), jnp.bfloat16),
            pltpu.VMEM((N_DEV, ROWS, CTX_LOC), jnp.bfloat16),
            pltpu.VMEM((ROWS, D_MODEL), jnp.float32),
            pltpu.SemaphoreType.DMA((N_DEV - 1,)),
            pltpu.SemaphoreType.DMA((N_DEV - 1,)),
        ],
        compiler_params=pltpu.CompilerParams(collective_id=0),
    )(x2, wq_loc, k_t, v_t, Wo)
    return out.reshape(B, SQ, D_MODEL)


# device time: 20428 ns/iter; 1.7402x vs baseline; 1.1978x over previous
import jax
import jax.numpy as jnp
from jax import lax
from jax.experimental import pallas as pl
from jax.experimental.pallas import tpu as pltpu

N_DEV = 8
B, SQ, SKV, DH = 2, 128, 128, 64
H_LOC = 4
CTX_LOC = H_LOC * DH
ROWS = B * SQ
D_MODEL = 512
BLK = 64
KCOLS = H_LOC * B * SKV
RB = ROWS // N_DEV
D_HID = N_DEV * CTX_LOC


def kernel(x, Wq, K_ext, V_ext, Wo):
    my_out = lax.axis_index("i")

    x2 = x.reshape(ROWS, D_MODEL).astype(jnp.bfloat16)
    wq_loc = (
        lax.dynamic_slice(Wq, (0, my_out * CTX_LOC), (D_MODEL, CTX_LOC))
        * 0.125
    ).astype(jnp.bfloat16)
    k_heads = K_ext.transpose(2, 3, 0, 1).reshape(H_LOC, DH, B * SKV)
    v_heads = V_ext.transpose(2, 0, 1, 3).reshape(H_LOC, B * SKV, DH)

    def body(x_ref, wq_ref, k_ref, v_ref, wo_ref, out_ref,
             kbd_ref, vbd_ref, ctx_ref, strip_ref, wobf_ref,
             p1_ref, p2_ref, s1_send, s1_recv, s2_send, s2_recv):
        my = lax.axis_index("i")

        barrier = pltpu.get_barrier_semaphore()
        for d in range(1, N_DEV):
            pl.semaphore_signal(
                barrier, inc=1,
                device_id=(lax.rem(my + d, N_DEV),),
                device_id_type=pl.DeviceIdType.MESH,
            )
        pl.semaphore_wait(barrier, N_DEV - 1)

        kbd_ref[...] = jnp.zeros((CTX_LOC, KCOLS), jnp.bfloat16)
        vbd_ref[...] = jnp.zeros((KCOLS, CTX_LOC), jnp.bfloat16)
        for h in range(H_LOC):
            kbd_ref[h * DH:(h + 1) * DH,
                    h * B * SKV:(h + 1) * B * SKV] = (
                k_ref[h].astype(jnp.bfloat16))
            vbd_ref[h * B * SKV:(h + 1) * B * SKV,
                    h * DH:(h + 1) * DH] = (
                v_ref[h].astype(jnp.bfloat16))

        q = jnp.dot(x_ref[...], wq_ref[...],
                    preferred_element_type=jnp.float32).astype(jnp.bfloat16)
        s = jnp.dot(q, kbd_ref[...], preferred_element_type=jnp.float32)

        r = lax.broadcasted_iota(jnp.int32, (ROWS, KCOLS), 0)
        c = lax.broadcasted_iota(jnp.int32, (ROWS, KCOLS), 1)
        r_b = r // SQ
        c_b = lax.rem(c, B * SKV) // SKV
        qb = lax.rem(r, SQ) // BLK
        kb = lax.rem(c, SKV) // BLK
        keep = (r_b == c_b) & (
            (qb == kb) | (kb == 0) | (lax.rem(qb + kb, 3) == 0))
        s = jnp.where(keep, s, -1e9)

        s_max = jnp.max(s, axis=1, keepdims=True)
        e = jnp.exp(s - s_max)
        parts = []
        for h in range(H_LOC):
            seg = e[:, h * B * SKV:(h + 1) * B * SKV]
            parts.append(seg / jnp.sum(seg, axis=1, keepdims=True))
        w = jnp.concatenate(parts, axis=1)

        ctx_ref[...] = jnp.dot(
            w.astype(jnp.bfloat16), vbd_ref[...],
            preferred_element_type=jnp.float32).astype(jnp.bfloat16)

        rdmas1 = []
        for d in range(1, N_DEV):
            t = lax.rem(my + d, N_DEV)
            rdma = pltpu.make_async_remote_copy(
                src_ref=ctx_ref.at[pl.ds(t * RB, RB), :],
                dst_ref=p1_ref.at[my],
                send_sem=s1_send.at[d - 1],
                recv_sem=s1_recv.at[d - 1],
                device_id=(t,),
                device_id_type=pl.DeviceIdType.MESH,
            )
            rdma.start()
            rdmas1.append(rdma)
        p1_ref[my] = ctx_ref[pl.ds(my * RB, RB), :]

        wobf_ref[...] = wo_ref[...].astype(jnp.bfloat16)

        for rdma in rdmas1:
            rdma.wait_recv()

        for o in range(N_DEV):
            strip_ref[:, o * CTX_LOC:(o + 1) * CTX_LOC] = p1_ref[o]
        blk = jnp.dot(strip_ref[...], wobf_ref[...],
                      preferred_element_type=jnp.float32)
        p2_ref[my] = blk.astype(jnp.bfloat16)

        rdmas2 = []
        for d in range(1, N_DEV):
            rdma = pltpu.make_async_remote_copy(
                src_ref=p2_ref.at[my],
                dst_ref=p2_ref.at[my],
                send_sem=s2_send.at[d - 1],
                recv_sem=s2_recv.at[d - 1],
                device_id=(lax.rem(my + d, N_DEV),),
                device_id_type=pl.DeviceIdType.MESH,
            )
            rdma.start()
            rdmas2.append(rdma)

        out_ref[pl.ds(my * RB, RB), :] = blk
        for d in range(1, N_DEV):
            rdmas2[d - 1].wait_recv()
            o = lax.rem(my - d + N_DEV, N_DEV)
            out_ref[pl.ds(o * RB, RB), :] = p2_ref[o].astype(jnp.float32)

        for rdma in rdmas1 + rdmas2:
            rdma.wait_send()

    out = pl.pallas_call(
        body,
        out_shape=jax.ShapeDtypeStruct((ROWS, D_MODEL), jnp.float32),
        in_specs=[pl.BlockSpec(memory_space=pltpu.VMEM)] * 5,
        out_specs=pl.BlockSpec(memory_space=pltpu.VMEM),
        scratch_shapes=[
            pltpu.VMEM((CTX_LOC, KCOLS), jnp.bfloat16),
            pltpu.VMEM((KCOLS, CTX_LOC), jnp.bfloat16),
            pltpu.VMEM((ROWS, CTX_LOC), jnp.bfloat16),
            pltpu.VMEM((RB, D_HID), jnp.bfloat16),
            pltpu.VMEM((D_HID, D_MODEL), jnp.bfloat16),
            pltpu.VMEM((N_DEV, RB, CTX_LOC), jnp.bfloat16),
            pltpu.VMEM((N_DEV, RB, D_MODEL), jnp.bfloat16),
            pltpu.SemaphoreType.DMA((N_DEV - 1,)),
            pltpu.SemaphoreType.DMA((N_DEV - 1,)),
            pltpu.SemaphoreType.DMA((N_DEV - 1,)),
            pltpu.SemaphoreType.DMA((N_DEV - 1,)),
        ],
        compiler_params=pltpu.CompilerParams(collective_id=0),
    )(x2, wq_loc, k_heads, v_heads, Wo)
    return out.reshape(B, SQ, D_MODEL)
